# baseline (speedup 1.0000x reference)
import functools

import jax
import jax.numpy as jnp
from jax import lax
from jax.experimental import pallas as pl
from jax.experimental.pallas import tpu as pltpu
from jax.experimental.pallas import tpu_sc as plsc

D = 1024
H = 73
R = 4 * 2048
SCR = 1024
NC, NS = 2, 16
NW = NC * NS
RPW = SCR // NW
CH = 16
CHW = CH * D
NCHUNK = RPW // CH
HSTR = R
HWB = H * RPW

_mesh = plsc.VectorSubcoreMesh(
    core_axis_name="c", subcore_axis_name="s", num_cores=NC, num_subcores=NS
)


@functools.partial(
    pl.kernel,
    out_type=jax.ShapeDtypeStruct((SCR * D,), jnp.float32),
    mesh=_mesh,
    scratch_types=[
        pltpu.VMEM((H * D,), jnp.float32),
        pltpu.VMEM((2 * CHW,), jnp.float32),
        pltpu.VMEM((HWB,), jnp.float32),
        pltpu.VMEM((RPW,), jnp.int32),
        pltpu.VMEM_SHARED((H * D,), jnp.float32),
        pltpu.SemaphoreType.DMA,
        pltpu.SemaphoreType.DMA,
        pltpu.SemaphoreType.DMA,
        pltpu.SemaphoreType.DMA,
        pltpu.SemaphoreType.DMA,
        pltpu.SemaphoreType.DMA,
    ],
    compiler_params=pltpu.CompilerParams(needs_layout_passes=False),
)
def _sc_add_pe(x_hbm, hour_hbm, pe_hbm, out_hbm, pe_v, xb_v, hbuf, idx_v,
               pe_sh, pe_sem, hr_sem, in0_sem, in1_sem, out0_sem, out1_sem):
    wid = lax.axis_index("s") * NC + lax.axis_index("c")
    row0 = wid * RPW
    io = lax.broadcasted_iota(jnp.int32, (16,), 0)
    in_sems = (in0_sem, in1_sem)
    out_sems = (out0_sem, out1_sem)

    def in_copy(c, p):
        return pltpu.make_async_copy(
            x_hbm.at[pl.ds((row0 + c * CH) * D, CHW)],
            xb_v.at[pl.ds(p * CHW, CHW)], in_sems[p])

    def out_copy(c, p):
        return pltpu.make_async_copy(
            xb_v.at[pl.ds(p * CHW, CHW)],
            out_hbm.at[pl.ds((row0 + c * CH) * D, CHW)], out_sems[p])

    def hour_dma(h, carry):
        src0 = h * HSTR + (wid // 4) * 512 + (wid % 4) * 32
        pltpu.async_copy(hour_hbm.at[pl.ds(src0, RPW)],
                         hbuf.at[pl.ds(h * RPW, RPW)], hr_sem)
        return carry

    lax.fori_loop(0, H, hour_dma, 0)
    in_copy(0, 0).start()
    in_copy(1, 1).start()

    @pl.when(lax.axis_index("s") == 0)
    def _stage_pe():
        pltpu.sync_copy(pe_hbm, pe_sh)

    plsc.subcore_barrier()
    pltpu.async_copy(pe_sh, pe_v, pe_sem)

    pltpu.make_async_copy(hour_hbm.at[pl.ds(0, HWB)], hbuf, hr_sem).wait()

    def amax_group(g, carry):
        col = g * 16
        best = hbuf[pl.ds(col, 16)]
        besti = io * 0
        for h in range(1, H):
            v = hbuf[pl.ds(h * RPW + col, 16)]
            upd = v > best
            besti = jnp.where(upd, h, besti)
            best = jnp.where(upd, v, best)
        idx_v[pl.ds(col, 16)] = besti * D
        return carry

    lax.fori_loop(0, RPW // 16, amax_group, 0)
    pltpu.make_async_copy(pe_hbm, pe_v, pe_sem).wait()

    def compute(c, p):
        xbase = p * CHW
        ivec = idx_v[pl.ds(c * CH, 16)]

        def row_body(j, rcarry):
            pb = jnp.max(jnp.where(io == j, ivec, 0))
            tl = j // 8
            rr = j - tl * 8
            xoff = xbase + tl * 8192 + rr * 128
            for ct in range(8):
                for k in range(8):
                    pv = pe_v[pl.ds(pb + ct * 128 + k * 16, 16)]
                    plsc.addupdate(
                        xb_v.at[pl.ds(xoff + ct * 1024 + k * 16, 16)], pv)
            return rcarry

        lax.fori_loop(0, CH, row_body, 0)

    for c in range(NCHUNK):
        p = c & 1
        q = p ^ 1
        if c + 1 < NCHUNK:
            if c >= 1:
                out_copy(c - 1, q).wait()
            if c + 1 >= 2:
                in_copy(c + 1, q).start()
        in_copy(c, p).wait()
        compute(c, p)
        out_copy(c, p).start()

    out_copy(NCHUNK - 2, 0).wait()
    out_copy(NCHUNK - 1, 1).wait()


TBLK = 256
TGRID = (R - SCR) // TBLK


def _tc_body(x_ref, h_ref, pe_ref, o_ref):
    hv = h_ref[...]
    lane = lax.broadcasted_iota(jnp.int32, (TBLK, H), 1)
    m = jnp.max(hv, axis=1, keepdims=True)
    cand = jnp.where(hv == m, lane, H)
    idx = jnp.min(cand, axis=1, keepdims=True)
    oh = (idx == lane).astype(jnp.bfloat16)
    acc = lax.dot_general(oh, pe_ref[...].astype(jnp.bfloat16),
                          (((1,), (0,)), ((), ())),
                          preferred_element_type=jnp.float32)
    o_ref[...] = x_ref[...] + acc


_tc_call = pl.pallas_call(
    _tc_body,
    grid=(TGRID,),
    in_specs=[
        pl.BlockSpec((TBLK, D), lambda i: (i + SCR // TBLK, 0)),
        pl.BlockSpec((TBLK, H), lambda i: (i + SCR // TBLK, 0)),
        pl.BlockSpec((H, D), lambda i: (0, 0)),
    ],
    out_specs=pl.BlockSpec((TBLK, D), lambda i: (i + SCR // TBLK, 0)),
    out_shape=jax.ShapeDtypeStruct((R, D), jnp.float32),
)


def kernel(x, hour_onehot, pe):
    xf = x.reshape(1024, 8, 8, 128).transpose(0, 2, 1, 3).reshape(-1)
    hf = hour_onehot.reshape(4, 16, 128, H).transpose(3, 1, 0, 2).reshape(-1)
    pf = pe.reshape(-1)
    sc_flat = _sc_add_pe(xf, hf, pf)
    sc2d = sc_flat.reshape(SCR // 8, 8, 8, 128).transpose(0, 2, 1, 3)
    sc2d = sc2d.reshape(SCR, D)

    x2 = x.reshape(R, D)
    h2 = hour_onehot.reshape(R, H)
    pe2 = pe.reshape(H, D)
    tc_out = _tc_call(x2, h2, pe2)

    out = lax.dynamic_update_slice(tc_out, sc2d, (0, 0))
    return out.reshape(x.shape)


# speedup vs baseline: 1.2021x; 1.2021x over previous
import functools

import jax
import jax.numpy as jnp
from jax import lax
from jax.experimental import pallas as pl
from jax.experimental.pallas import tpu as pltpu
from jax.experimental.pallas import tpu_sc as plsc

D = 1024
H = 73
R = 4 * 2048
SCR = 512
NC, NS = 2, 16
NW = NC * NS
RPW = SCR // NW
CH = 8
CHW = CH * D
NCHUNK = RPW // CH
HSTR = R
HWB = H * RPW

_mesh = plsc.VectorSubcoreMesh(
    core_axis_name="c", subcore_axis_name="s", num_cores=NC, num_subcores=NS
)


@functools.partial(
    pl.kernel,
    out_type=jax.ShapeDtypeStruct((SCR * D,), jnp.float32),
    mesh=_mesh,
    scratch_types=[
        pltpu.VMEM((H * D,), jnp.float32),
        pltpu.VMEM((2 * CHW,), jnp.float32),
        pltpu.VMEM((HWB,), jnp.float32),
        pltpu.VMEM((RPW,), jnp.int32),
        pltpu.VMEM_SHARED((H * D,), jnp.float32),
        pltpu.SemaphoreType.DMA,
        pltpu.SemaphoreType.DMA,
        pltpu.SemaphoreType.DMA,
        pltpu.SemaphoreType.DMA,
        pltpu.SemaphoreType.DMA,
        pltpu.SemaphoreType.DMA,
    ],
    compiler_params=pltpu.CompilerParams(needs_layout_passes=False),
)
def _sc_add_pe(x_hbm, hour_hbm, pe_hbm, out_hbm, pe_v, xb_v, hbuf, idx_v,
               pe_sh, pe_sem, hr_sem, in0_sem, in1_sem, out0_sem, out1_sem):
    wid = lax.axis_index("s") * NC + lax.axis_index("c")
    row0 = wid * RPW
    io = lax.broadcasted_iota(jnp.int32, (16,), 0)
    in_sems = (in0_sem, in1_sem)
    out_sems = (out0_sem, out1_sem)

    def in_copy(c, p):
        return pltpu.make_async_copy(
            x_hbm.at[pl.ds((row0 + c * CH) * D, CHW)],
            xb_v.at[pl.ds(p * CHW, CHW)], in_sems[p])

    def out_copy(c, p):
        return pltpu.make_async_copy(
            xb_v.at[pl.ds(p * CHW, CHW)],
            out_hbm.at[pl.ds((row0 + c * CH) * D, CHW)], out_sems[p])

    def hour_dma(h, carry):
        l0 = wid * RPW
        src0 = h * HSTR + (l0 // 128) * 512 + l0 % 128
        pltpu.async_copy(hour_hbm.at[pl.ds(src0, RPW)],
                         hbuf.at[pl.ds(h * RPW, RPW)], hr_sem)
        return carry

    lax.fori_loop(0, H, hour_dma, 0)
    in_copy(0, 0).start()
    in_copy(1, 1).start()

    @pl.when(lax.axis_index("s") == 0)
    def _stage_pe():
        pltpu.sync_copy(pe_hbm, pe_sh)

    plsc.subcore_barrier()
    pltpu.async_copy(pe_sh, pe_v, pe_sem)

    pltpu.make_async_copy(hour_hbm.at[pl.ds(0, HWB)], hbuf, hr_sem).wait()

    def amax_group(g, carry):
        col = g * 16
        best = hbuf[pl.ds(col, 16)]
        besti = io * 0
        for h in range(1, H):
            v = hbuf[pl.ds(h * RPW + col, 16)]
            upd = v > best
            besti = jnp.where(upd, h, besti)
            best = jnp.where(upd, v, best)
        idx_v[pl.ds(col, 16)] = besti * D
        return carry

    lax.fori_loop(0, RPW // 16, amax_group, 0)
    pltpu.make_async_copy(pe_hbm, pe_v, pe_sem).wait()

    def compute(c, p):
        xbase = p * CHW
        g16 = (c * CH) // 16 * 16
        lo = c * CH - g16
        ivec = idx_v[pl.ds(g16, 16)]

        def row_body(j, rcarry):
            pb = jnp.max(jnp.where(io == j + lo, ivec, 0))
            tl = j // 8
            rr = j - tl * 8
            xoff = xbase + tl * 8192 + rr * 128
            for ct in range(8):
                for k in range(8):
                    pv = pe_v[pl.ds(pb + ct * 128 + k * 16, 16)]
                    plsc.addupdate(
                        xb_v.at[pl.ds(xoff + ct * 1024 + k * 16, 16)], pv)
            return rcarry

        lax.fori_loop(0, CH, row_body, 0)

    for c in range(NCHUNK):
        p = c & 1
        q = p ^ 1
        if c + 1 < NCHUNK:
            if c >= 1:
                out_copy(c - 1, q).wait()
            if c + 1 >= 2:
                in_copy(c + 1, q).start()
        in_copy(c, p).wait()
        compute(c, p)
        out_copy(c, p).start()

    out_copy(NCHUNK - 2, 0).wait()
    out_copy(NCHUNK - 1, 1).wait()


TBLK = 512
TGRID = (R - SCR) // TBLK


def _tc_body(x_ref, h_ref, pe_ref, o_ref):
    hv = h_ref[...]
    lane = lax.broadcasted_iota(jnp.int32, (TBLK, H), 1)
    m = jnp.max(hv, axis=1, keepdims=True)
    cand = jnp.where(hv == m, lane, H)
    idx = jnp.min(cand, axis=1, keepdims=True)
    oh = (idx == lane).astype(jnp.bfloat16)
    acc = lax.dot_general(oh, pe_ref[...].astype(jnp.bfloat16),
                          (((1,), (0,)), ((), ())),
                          preferred_element_type=jnp.float32)
    o_ref[...] = x_ref[...] + acc


_tc_call = pl.pallas_call(
    _tc_body,
    grid=(TGRID,),
    in_specs=[
        pl.BlockSpec((TBLK, D), lambda i: (i + SCR // TBLK, 0)),
        pl.BlockSpec((TBLK, H), lambda i: (i + SCR // TBLK, 0)),
        pl.BlockSpec((H, D), lambda i: (0, 0)),
    ],
    out_specs=pl.BlockSpec((TBLK, D), lambda i: (i + SCR // TBLK, 0)),
    out_shape=jax.ShapeDtypeStruct((R, D), jnp.float32),
)


def kernel(x, hour_onehot, pe):
    xf = x.reshape(1024, 8, 8, 128).transpose(0, 2, 1, 3).reshape(-1)
    hf = hour_onehot.reshape(4, 16, 128, H).transpose(3, 1, 0, 2).reshape(-1)
    pf = pe.reshape(-1)
    sc_flat = _sc_add_pe(xf, hf, pf)
    sc2d = sc_flat.reshape(SCR // 8, 8, 8, 128).transpose(0, 2, 1, 3)
    sc2d = sc2d.reshape(SCR, D)

    x2 = x.reshape(R, D)
    h2 = hour_onehot.reshape(R, H)
    pe2 = pe.reshape(H, D)
    tc_out = _tc_call(x2, h2, pe2)

    out = lax.dynamic_update_slice(tc_out, sc2d, (0, 0))
    return out.reshape(x.shape)
